# Initial kernel scaffold; baseline (speedup 1.0000x reference)
import jax
import jax.numpy as jnp
from jax.experimental import pallas as pl


def kernel(x, noise_idx):
    raise NotImplementedError("write your pallas kernel here")


# speedup vs baseline: 4.4934x; 4.4934x over previous
import jax
import jax.numpy as jnp
from jax import lax
from jax.experimental import pallas as pl
from jax.experimental.pallas import tpu as pltpu
from jax.experimental.pallas import tpu_sc as plsc

B, D = 8192, 4096
N = 408
HALF = N // 2
L = 16
CH = 8
NW = 32
ROWS_PER = B // NW
GROUPS = CH * N // L


def _body(x_hbm, k_hbm, v_hbm, n_hbm, out_hbm, xbuf, kbuf, vbuf, nbuf):
    nc = 2
    wid = lax.axis_index("s") * nc + lax.axis_index("c")
    base = wid * ROWS_PER

    def chunk(ci, carry):
        r0 = base + ci * CH
        pltpu.sync_copy(x_hbm.at[pl.ds(r0 * D, CH * D)], xbuf)
        pltpu.sync_copy(k_hbm.at[pl.ds(r0 * N, CH * N)], kbuf)
        pltpu.sync_copy(v_hbm.at[pl.ds(r0 * N, CH * N)], vbuf)
        pltpu.sync_copy(n_hbm.at[pl.ds(r0 * N, CH * N)], nbuf)
        kbase = r0 * D
        for g in range(GROUPS):
            k = kbuf[pl.ds(g * L, L)]
            nk = nbuf[pl.ds(g * L, L)]
            v = vbuf[pl.ds(g * L, L)]
            plsc.store_scatter(xbuf, [k - kbase], v, mask=k != nk)
        pltpu.sync_copy(xbuf, out_hbm.at[pl.ds(r0 * D, CH * D)])
        return carry

    lax.fori_loop(0, ROWS_PER // CH, chunk, 0)


@jax.jit
def kernel(x, noise_idx):
    keys = (jnp.arange(B, dtype=jnp.int32)[:, None] * D + noise_idx).reshape(B * N)
    vals = jnp.where(
        jnp.arange(N, dtype=jnp.int32)[None, :] < HALF, 1.0, -1.0
    ).astype(jnp.float32)
    vals = jnp.broadcast_to(vals, (B, N)).reshape(B * N)
    skeys, svals = lax.sort_key_val(keys, vals, is_stable=False)
    snext = jnp.concatenate(
        [skeys[1:], jnp.full((1,), jnp.iinfo(jnp.int32).min, jnp.int32)]
    )

    mesh = plsc.VectorSubcoreMesh(core_axis_name="c", subcore_axis_name="s")
    run = pl.kernel(
        _body,
        out_type=jax.ShapeDtypeStruct((B * D,), jnp.float32),
        mesh=mesh,
        scratch_types=[
            pltpu.VMEM((CH * D,), jnp.float32),
            pltpu.VMEM((CH * N,), jnp.int32),
            pltpu.VMEM((CH * N,), jnp.float32),
            pltpu.VMEM((CH * N,), jnp.int32),
        ],
        compiler_params=pltpu.CompilerParams(
            use_tc_tiling_on_sc=False, needs_layout_passes=False
        ),
    )
    return run(x.reshape(B * D), skeys, svals, snext).reshape(B, D)


# speedup vs baseline: 4.5445x; 1.0114x over previous
import jax
import jax.numpy as jnp
from jax import lax
from jax.experimental import pallas as pl
from jax.experimental.pallas import tpu as pltpu
from jax.experimental.pallas import tpu_sc as plsc

B, D = 8192, 4096
N = 408
HALF = N // 2
L = 16
CH = 8
NW = 32
ROWS_PER = B // NW
GROUPS = CH * N // L


def _body(x_hbm, k_hbm, v_hbm, out_hbm, xbuf, kbuf, vbuf):
    nc = 2
    wid = lax.axis_index("s") * nc + lax.axis_index("c")
    base = wid * ROWS_PER

    lane = lax.iota(jnp.int32, L)
    kbuf[pl.ds(CH * N, L)] = lane * 0 - 1

    def chunk(ci, carry):
        r0 = base + ci * CH
        pltpu.sync_copy(x_hbm.at[pl.ds(r0 * D, CH * D)], xbuf)
        pltpu.sync_copy(k_hbm.at[pl.ds(r0 * N, CH * N)], kbuf.at[pl.ds(0, CH * N)])
        pltpu.sync_copy(v_hbm.at[pl.ds(r0 * N, CH * N)], vbuf)
        kbase = r0 * D
        for g in range(GROUPS):
            k = kbuf[pl.ds(g * L, L)]
            nk = kbuf[pl.ds(g * L + 1, L)]
            v = vbuf[pl.ds(g * L, L)]
            plsc.store_scatter(xbuf, [k - kbase], v, mask=k != nk)
        pltpu.sync_copy(xbuf, out_hbm.at[pl.ds(r0 * D, CH * D)])
        return carry

    lax.fori_loop(0, ROWS_PER // CH, chunk, 0)


@jax.jit
def kernel(x, noise_idx):
    keys = (jnp.arange(B, dtype=jnp.int32)[:, None] * D + noise_idx).reshape(B * N)
    vals = jnp.where(
        jnp.arange(N, dtype=jnp.int32)[None, :] < HALF, 1.0, -1.0
    ).astype(jnp.float32)
    vals = jnp.broadcast_to(vals, (B, N)).reshape(B * N)
    skeys, svals = lax.sort_key_val(keys, vals, is_stable=False)

    mesh = plsc.VectorSubcoreMesh(core_axis_name="c", subcore_axis_name="s")
    run = pl.kernel(
        _body,
        out_type=jax.ShapeDtypeStruct((B * D,), jnp.float32),
        mesh=mesh,
        scratch_types=[
            pltpu.VMEM((CH * D,), jnp.float32),
            pltpu.VMEM((CH * N + L,), jnp.int32),
            pltpu.VMEM((CH * N,), jnp.float32),
        ],
        compiler_params=pltpu.CompilerParams(
            use_tc_tiling_on_sc=False, needs_layout_passes=False
        ),
    )
    return run(x.reshape(B * D), skeys, svals).reshape(B, D)


# speedup vs baseline: 4.6500x; 1.0232x over previous
import jax
import jax.numpy as jnp
from jax import lax
from jax.experimental import pallas as pl
from jax.experimental.pallas import tpu as pltpu
from jax.experimental.pallas import tpu_sc as plsc

B, D = 8192, 4096
N = 408
HALF = N // 2
L = 16
CH = 8
NW = 32
ROWS_PER = B // NW
GROUPS = CH * N // L


NCHUNK = ROWS_PER // CH
NPAIR = NCHUNK // 2


def _body(
    x_hbm, k_hbm, v_hbm, out_hbm,
    xbuf0, xbuf1, kbuf0, kbuf1, vbuf0, vbuf1,
    sem_in0, sem_in1, sem_out0, sem_out1,
):
    nc = 2
    wid = lax.axis_index("s") * nc + lax.axis_index("c")
    base = wid * ROWS_PER
    xbufs, kbufs, vbufs = (xbuf0, xbuf1), (kbuf0, kbuf1), (vbuf0, vbuf1)
    sem_in, sem_out = (sem_in0, sem_in1), (sem_out0, sem_out1)

    lane = lax.iota(jnp.int32, L)
    kbuf0[pl.ds(CH * N, L)] = lane * 0 - 1
    kbuf1[pl.ds(CH * N, L)] = lane * 0 - 1

    def issue_in(c, s):
        r0 = base + c * CH
        pltpu.async_copy(x_hbm.at[pl.ds(r0 * D, CH * D)], xbufs[s], sem_in[s])
        pltpu.async_copy(
            k_hbm.at[pl.ds(r0 * N, CH * N)], kbufs[s].at[pl.ds(0, CH * N)], sem_in[s]
        )
        pltpu.async_copy(v_hbm.at[pl.ds(r0 * N, CH * N)], vbufs[s], sem_in[s])

    def wait_in(c, s):
        r0 = base + c * CH
        pltpu.make_async_copy(
            x_hbm.at[pl.ds(r0 * D, CH * D)], xbufs[s], sem_in[s]
        ).wait()
        pltpu.make_async_copy(
            k_hbm.at[pl.ds(r0 * N, CH * N)], kbufs[s].at[pl.ds(0, CH * N)], sem_in[s]
        ).wait()
        pltpu.make_async_copy(
            v_hbm.at[pl.ds(r0 * N, CH * N)], vbufs[s], sem_in[s]
        ).wait()

    def wait_out(c, s):
        r0 = base + c * CH
        pltpu.make_async_copy(
            xbufs[s], out_hbm.at[pl.ds(r0 * D, CH * D)], sem_out[s]
        ).wait()

    issue_in(0, 0)
    issue_in(1, 1)

    def pair(i, carry):
        for s in (0, 1):
            c = 2 * i + s
            r0 = base + c * CH
            wait_in(c, s)
            kbase = r0 * D
            kbuf, vbuf, xbuf = kbufs[s], vbufs[s], xbufs[s]
            for g in range(GROUPS):
                k = kbuf[pl.ds(g * L, L)]
                nk = kbuf[pl.ds(g * L + 1, L)]
                v = vbuf[pl.ds(g * L, L)]
                plsc.store_scatter(xbuf, [k - kbase], v, mask=k != nk)
            pltpu.async_copy(xbuf, out_hbm.at[pl.ds(r0 * D, CH * D)], sem_out[s])

        @pl.when(i < NPAIR - 1)
        def _prefetch():
            for s in (0, 1):
                wait_out(2 * i + s, s)
                issue_in(2 * i + s + 2, s)

        return carry

    lax.fori_loop(0, NPAIR, pair, 0)
    wait_out(NCHUNK - 2, 0)
    wait_out(NCHUNK - 1, 1)


@jax.jit
def kernel(x, noise_idx):
    keys = (jnp.arange(B, dtype=jnp.int32)[:, None] * D + noise_idx).reshape(B * N)
    vals = jnp.where(
        jnp.arange(N, dtype=jnp.int32)[None, :] < HALF, 1.0, -1.0
    ).astype(jnp.float32)
    vals = jnp.broadcast_to(vals, (B, N)).reshape(B * N)
    skeys, svals = lax.sort_key_val(keys, vals, is_stable=False)

    mesh = plsc.VectorSubcoreMesh(core_axis_name="c", subcore_axis_name="s")
    run = pl.kernel(
        _body,
        out_type=jax.ShapeDtypeStruct((B * D,), jnp.float32),
        mesh=mesh,
        scratch_types=[
            pltpu.VMEM((CH * D,), jnp.float32),
            pltpu.VMEM((CH * D,), jnp.float32),
            pltpu.VMEM((CH * N + L,), jnp.int32),
            pltpu.VMEM((CH * N + L,), jnp.int32),
            pltpu.VMEM((CH * N,), jnp.float32),
            pltpu.VMEM((CH * N,), jnp.float32),
            pltpu.SemaphoreType.DMA,
            pltpu.SemaphoreType.DMA,
            pltpu.SemaphoreType.DMA,
            pltpu.SemaphoreType.DMA,
        ],
        compiler_params=pltpu.CompilerParams(
            use_tc_tiling_on_sc=False, needs_layout_passes=False
        ),
    )
    return run(x.reshape(B * D), skeys, svals).reshape(B, D)


# speedup vs baseline: 4.6857x; 1.0077x over previous
import jax
import jax.numpy as jnp
from jax import lax
from jax.experimental import pallas as pl
from jax.experimental.pallas import tpu as pltpu
from jax.experimental.pallas import tpu_sc as plsc

B, D = 8192, 4096
N = 408
HALF = N // 2
L = 16
CH = 8
NW = 32
ROWS_PER = B // NW
GROUPS = CH * N // L


NCHUNK = ROWS_PER // CH
NPAIR = NCHUNK // 2


def _body(
    x_hbm, k_hbm, v_hbm, out_hbm,
    xbuf0, xbuf1, kbuf0, kbuf1, vbuf0, vbuf1,
    sem_in0, sem_in1, sem_out0, sem_out1,
):
    nc = 2
    wid = lax.axis_index("s") * nc + lax.axis_index("c")
    base = wid * ROWS_PER
    xbufs, kbufs, vbufs = (xbuf0, xbuf1), (kbuf0, kbuf1), (vbuf0, vbuf1)
    sem_in, sem_out = (sem_in0, sem_in1), (sem_out0, sem_out1)

    lane = lax.iota(jnp.int32, L)
    kbuf0[pl.ds(CH * N, L)] = lane * 0 - 1
    kbuf1[pl.ds(CH * N, L)] = lane * 0 - 1

    def issue_in(c, s):
        r0 = base + c * CH
        pltpu.async_copy(x_hbm.at[pl.ds(r0 * D, CH * D)], xbufs[s], sem_in[s])
        pltpu.async_copy(
            k_hbm.at[pl.ds(r0 * N, CH * N)], kbufs[s].at[pl.ds(0, CH * N)], sem_in[s]
        )
        pltpu.async_copy(v_hbm.at[pl.ds(r0 * N, CH * N)], vbufs[s], sem_in[s])

    def wait_in(c, s):
        r0 = base + c * CH
        pltpu.make_async_copy(
            x_hbm.at[pl.ds(r0 * D, CH * D)], xbufs[s], sem_in[s]
        ).wait()
        pltpu.make_async_copy(
            k_hbm.at[pl.ds(r0 * N, CH * N)], kbufs[s].at[pl.ds(0, CH * N)], sem_in[s]
        ).wait()
        pltpu.make_async_copy(
            v_hbm.at[pl.ds(r0 * N, CH * N)], vbufs[s], sem_in[s]
        ).wait()

    def wait_out(c, s):
        r0 = base + c * CH
        pltpu.make_async_copy(
            xbufs[s], out_hbm.at[pl.ds(r0 * D, CH * D)], sem_out[s]
        ).wait()

    issue_in(0, 0)
    issue_in(1, 1)

    def pair(i, carry):
        for s in (0, 1):
            c = 2 * i + s
            r0 = base + c * CH
            wait_in(c, s)
            kbase = r0 * D
            kbuf, vbuf, xbuf = kbufs[s], vbufs[s], xbufs[s]
            for g in range(GROUPS):
                k = kbuf[pl.ds(g * L, L)]
                nk = kbuf[pl.ds(g * L + 1, L)]
                v = vbuf[pl.ds(g * L, L)]
                plsc.store_scatter(xbuf, [k - kbase], v, mask=k != nk)
            pltpu.async_copy(xbuf, out_hbm.at[pl.ds(r0 * D, CH * D)], sem_out[s])

        @pl.when(i < NPAIR - 1)
        def _prefetch():
            for s in (0, 1):
                wait_out(2 * i + s, s)
                issue_in(2 * i + s + 2, s)

        return carry

    lax.fori_loop(0, NPAIR, pair, 0)
    wait_out(NCHUNK - 2, 0)
    wait_out(NCHUNK - 1, 1)


@jax.jit
def kernel(x, noise_idx):
    flat = jnp.arange(B * N, dtype=jnp.int32)
    keys = noise_idx.reshape(B * N) + (flat // N) * D
    vals = jnp.where(flat % N < HALF, 1.0, -1.0).astype(jnp.float32)
    skeys, svals = lax.sort_key_val(keys, vals, is_stable=False)

    mesh = plsc.VectorSubcoreMesh(core_axis_name="c", subcore_axis_name="s")
    run = pl.kernel(
        _body,
        out_type=jax.ShapeDtypeStruct((B * D,), jnp.float32),
        mesh=mesh,
        scratch_types=[
            pltpu.VMEM((CH * D,), jnp.float32),
            pltpu.VMEM((CH * D,), jnp.float32),
            pltpu.VMEM((CH * N + L,), jnp.int32),
            pltpu.VMEM((CH * N + L,), jnp.int32),
            pltpu.VMEM((CH * N,), jnp.float32),
            pltpu.VMEM((CH * N,), jnp.float32),
            pltpu.SemaphoreType.DMA,
            pltpu.SemaphoreType.DMA,
            pltpu.SemaphoreType.DMA,
            pltpu.SemaphoreType.DMA,
        ],
        compiler_params=pltpu.CompilerParams(
            use_tc_tiling_on_sc=False, needs_layout_passes=False
        ),
    )
    return run(x.reshape(B * D), skeys, svals).reshape(B, D)
